# baseline (speedup 1.0000x reference)
import jax
import jax.numpy as jnp
from jax.experimental import pallas as pl

_TOKEN_BLOCK = 512


def _fused_router_kernel(x_ref, wm_ref, bm_ref, wr_ref, br_ref, out_ref):
    xb = x_ref[...].astype(jnp.bfloat16)
    h = jnp.dot(xb, wm_ref[...], preferred_element_type=jnp.float32)
    h = h + bm_ref[...]
    logits = jnp.dot(h.astype(jnp.bfloat16), wr_ref[...],
                     preferred_element_type=jnp.float32)
    logits = logits + br_ref[...]
    m = jnp.max(logits, axis=-1, keepdims=True)
    e = jnp.exp(logits - m)
    out_ref[...] = e / jnp.sum(e, axis=-1, keepdims=True)


def kernel(x, W_model, b_model, W_router, b_router):
    tokens, d_model = x.shape
    h_out = W_model.shape[1]
    n_experts = W_router.shape[1]
    tb = min(_TOKEN_BLOCK, tokens)
    W_model = W_model.astype(jnp.bfloat16)
    W_router = W_router.astype(jnp.bfloat16)
    bm = b_model.reshape(1, h_out)
    br = b_router.reshape(1, n_experts)
    return pl.pallas_call(
        _fused_router_kernel,
        grid=(tokens // tb,),
        in_specs=[
            pl.BlockSpec((tb, d_model), lambda i: (i, 0)),
            pl.BlockSpec((d_model, h_out), lambda i: (0, 0)),
            pl.BlockSpec((1, h_out), lambda i: (0, 0)),
            pl.BlockSpec((h_out, n_experts), lambda i: (0, 0)),
            pl.BlockSpec((1, n_experts), lambda i: (0, 0)),
        ],
        out_specs=pl.BlockSpec((tb, n_experts), lambda i: (i, 0)),
        out_shape=jax.ShapeDtypeStruct((tokens, n_experts), jnp.float32),
    )(x, W_model, bm, W_router, br)


# speedup vs baseline: 1.0685x; 1.0685x over previous
import jax
import jax.numpy as jnp
from jax.experimental import pallas as pl

_TOKEN_BLOCK = 512


def _fused_router_kernel(x_ref, wm_ref, bm_ref, wr_ref, br_ref, out_ref):
    h = jnp.dot(x_ref[...], wm_ref[...], preferred_element_type=jnp.float32)
    h = h + bm_ref[...]
    logits = jnp.dot(h, wr_ref[...], preferred_element_type=jnp.float32)
    logits = logits + br_ref[...]
    m = jnp.max(logits, axis=-1, keepdims=True)
    e = jnp.exp(logits - m)
    out_ref[...] = e / jnp.sum(e, axis=-1, keepdims=True)


def kernel(x, W_model, b_model, W_router, b_router):
    tokens, d_model = x.shape
    h_out = W_model.shape[1]
    n_experts = W_router.shape[1]
    tb = min(_TOKEN_BLOCK, tokens)
    bm = b_model.reshape(1, h_out)
    br = b_router.reshape(1, n_experts)
    return pl.pallas_call(
        _fused_router_kernel,
        grid=(tokens // tb,),
        in_specs=[
            pl.BlockSpec((tb, d_model), lambda i: (i, 0)),
            pl.BlockSpec((d_model, h_out), lambda i: (0, 0)),
            pl.BlockSpec((1, h_out), lambda i: (0, 0)),
            pl.BlockSpec((h_out, n_experts), lambda i: (0, 0)),
            pl.BlockSpec((1, n_experts), lambda i: (0, 0)),
        ],
        out_specs=pl.BlockSpec((tb, n_experts), lambda i: (i, 0)),
        out_shape=jax.ShapeDtypeStruct((tokens, n_experts), jnp.float32),
    )(x, W_model, bm, W_router, br)


# speedup vs baseline: 1.0968x; 1.0264x over previous
import jax
import jax.numpy as jnp
from jax.experimental import pallas as pl

_TOKEN_BLOCK = 1024


def _fused_router_kernel(x_ref, wm_ref, bm_ref, wr_ref, br_ref, out_ref):
    h = jnp.dot(x_ref[...], wm_ref[...], preferred_element_type=jnp.float32)
    h = h + bm_ref[...]
    logits = jnp.dot(h, wr_ref[...], preferred_element_type=jnp.float32)
    logits = logits + br_ref[...]
    m = jnp.max(logits, axis=-1, keepdims=True)
    e = jnp.exp(logits - m)
    out_ref[...] = e / jnp.sum(e, axis=-1, keepdims=True)


def kernel(x, W_model, b_model, W_router, b_router):
    tokens, d_model = x.shape
    h_out = W_model.shape[1]
    n_experts = W_router.shape[1]
    tb = min(_TOKEN_BLOCK, tokens)
    bm = b_model.reshape(1, h_out)
    br = b_router.reshape(1, n_experts)
    return pl.pallas_call(
        _fused_router_kernel,
        grid=(tokens // tb,),
        in_specs=[
            pl.BlockSpec((tb, d_model), lambda i: (i, 0)),
            pl.BlockSpec((d_model, h_out), lambda i: (0, 0)),
            pl.BlockSpec((1, h_out), lambda i: (0, 0)),
            pl.BlockSpec((h_out, n_experts), lambda i: (0, 0)),
            pl.BlockSpec((1, n_experts), lambda i: (0, 0)),
        ],
        out_specs=pl.BlockSpec((tb, n_experts), lambda i: (i, 0)),
        out_shape=jax.ShapeDtypeStruct((tokens, n_experts), jnp.float32),
    )(x, W_model, bm, W_router, br)


# speedup vs baseline: 2.8112x; 2.5632x over previous
import jax
import jax.numpy as jnp
from jax.experimental import pallas as pl
from jax.experimental.pallas import tpu as pltpu

_TOKEN_BLOCK = 1024


def _router_kernel(x_ref, wm_ref, bm_ref, wr_ref, br_ref, out_ref,
                   wf_ref, bf_ref):
    @pl.when(pl.program_id(0) == 0)
    def _fuse_weights():
        wf_ref[...] = jnp.dot(wm_ref[...], wr_ref[...],
                              preferred_element_type=jnp.float32)
        bf_ref[...] = jnp.dot(bm_ref[...], wr_ref[...],
                              preferred_element_type=jnp.float32) + br_ref[...]

    logits = jnp.dot(x_ref[...], wf_ref[...],
                     preferred_element_type=jnp.float32)
    logits = logits + bf_ref[...]
    m = jnp.max(logits, axis=-1, keepdims=True)
    e = jnp.exp(logits - m)
    out_ref[...] = e / jnp.sum(e, axis=-1, keepdims=True)


def kernel(x, W_model, b_model, W_router, b_router):
    tokens, d_model = x.shape
    h_out = W_model.shape[1]
    n_experts = W_router.shape[1]
    tb = min(_TOKEN_BLOCK, tokens)
    bm = b_model.reshape(1, h_out)
    br = b_router.reshape(1, n_experts)
    return pl.pallas_call(
        _router_kernel,
        grid=(tokens // tb,),
        in_specs=[
            pl.BlockSpec((tb, d_model), lambda i: (i, 0)),
            pl.BlockSpec((d_model, h_out), lambda i: (0, 0)),
            pl.BlockSpec((1, h_out), lambda i: (0, 0)),
            pl.BlockSpec((h_out, n_experts), lambda i: (0, 0)),
            pl.BlockSpec((1, n_experts), lambda i: (0, 0)),
        ],
        out_specs=pl.BlockSpec((tb, n_experts), lambda i: (i, 0)),
        out_shape=jax.ShapeDtypeStruct((tokens, n_experts), jnp.float32),
        scratch_shapes=[
            pltpu.VMEM((d_model, n_experts), jnp.float32),
            pltpu.VMEM((1, n_experts), jnp.float32),
        ],
    )(x, W_model, bm, W_router, br)


# speedup vs baseline: 2.8292x; 1.0064x over previous
import jax
import jax.numpy as jnp
from jax.experimental import pallas as pl
from jax.experimental.pallas import tpu as pltpu

_TOKEN_BLOCK = 2048


def _router_kernel(x_ref, wm_ref, bm_ref, wr_ref, br_ref, out_ref,
                   wf_ref, bf_ref):
    @pl.when(pl.program_id(0) == 0)
    def _fuse_weights():
        wf_ref[...] = jnp.dot(wm_ref[...], wr_ref[...],
                              preferred_element_type=jnp.float32)
        bf_ref[...] = jnp.dot(bm_ref[...], wr_ref[...],
                              preferred_element_type=jnp.float32) + br_ref[...]

    logits = jnp.dot(x_ref[...], wf_ref[...],
                     preferred_element_type=jnp.float32)
    logits = logits + bf_ref[...]
    m = jnp.max(logits, axis=-1, keepdims=True)
    e = jnp.exp(logits - m)
    out_ref[...] = e / jnp.sum(e, axis=-1, keepdims=True)


def kernel(x, W_model, b_model, W_router, b_router):
    tokens, d_model = x.shape
    h_out = W_model.shape[1]
    n_experts = W_router.shape[1]
    tb = min(_TOKEN_BLOCK, tokens)
    bm = b_model.reshape(1, h_out)
    br = b_router.reshape(1, n_experts)
    return pl.pallas_call(
        _router_kernel,
        grid=(tokens // tb,),
        in_specs=[
            pl.BlockSpec((tb, d_model), lambda i: (i, 0)),
            pl.BlockSpec((d_model, h_out), lambda i: (0, 0)),
            pl.BlockSpec((1, h_out), lambda i: (0, 0)),
            pl.BlockSpec((h_out, n_experts), lambda i: (0, 0)),
            pl.BlockSpec((1, n_experts), lambda i: (0, 0)),
        ],
        out_specs=pl.BlockSpec((tb, n_experts), lambda i: (i, 0)),
        out_shape=jax.ShapeDtypeStruct((tokens, n_experts), jnp.float32),
        scratch_shapes=[
            pltpu.VMEM((d_model, n_experts), jnp.float32),
            pltpu.VMEM((1, n_experts), jnp.float32),
        ],
    )(x, W_model, bm, W_router, br)
